# baseline (speedup 1.0000x reference)
import jax
import jax.numpy as jnp
from jax import lax
from jax.experimental import pallas as pl
from jax.experimental.pallas import tpu as pltpu
from jax.experimental.pallas import tpu_sc as plsc

NC = 2
NS = 16
NW = NC * NS

EMBED = 128
NBUF = 8


def _embed_body(tok_hbm, table_hbm, out_hbm, idx_v, bufs, gsem, wsem,
                seq_per_w, seq_len):
    wid = lax.axis_index("s") * NC + lax.axis_index("c")
    pltpu.sync_copy(tok_hbm.at[wid], idx_v)
    seq0 = wid * seq_per_w

    def g_copy(j, b):
        return pltpu.make_async_copy(
            table_hbm.at[idx_v.at[j, pl.ds(0, seq_len)]], bufs.at[b],
            gsem.at[b])

    def w_copy(j, b):
        return pltpu.make_async_copy(
            bufs.at[b], out_hbm.at[seq0 + j], wsem.at[b])

    for b in range(NBUF):
        g_copy(b, b).start()

    @pl.loop(0, seq_per_w, step=NBUF)
    def _(j0):
        for b in range(NBUF):
            j = j0 + b
            g_copy(j, b).wait()
            w_copy(j, b).start()

            @pl.when(j + NBUF < seq_per_w)
            def _():
                w_copy(j, b).wait()
                g_copy(j + NBUF, b).start()

    for b in range(NBUF):
        w_copy(seq_per_w - NBUF + b, b).wait()


NCHUNK = 4


def kernel(tokens, table):
    n_seq, seq_len = tokens.shape
    assert n_seq % (NW * NCHUNK) == 0
    ch_seq = n_seq // NCHUNK
    seq_per_w = ch_seq // NW
    assert seq_per_w % NBUF == 0
    tok_pad = jnp.zeros((n_seq, 128), jnp.int32)
    tok_pad = lax.dynamic_update_slice(
        tok_pad, tokens.astype(jnp.int32), (0, 0))
    tok_cubes = tok_pad.reshape(NCHUNK, NW, seq_per_w, 128)

    mesh = plsc.VectorSubcoreMesh(
        core_axis_name="c", subcore_axis_name="s",
        num_cores=NC, num_subcores=NS)

    def body(tok_hbm, table_hbm, out_hbm, idx_v, bufs, gsem, wsem):
        _embed_body(tok_hbm, table_hbm, out_hbm, idx_v, bufs, gsem, wsem,
                    seq_per_w, seq_len)

    call = pl.kernel(
        body,
        out_type=jax.ShapeDtypeStruct((ch_seq, seq_len, EMBED), jnp.float32),
        mesh=mesh,
        compiler_params=pltpu.CompilerParams(use_tc_tiling_on_sc=True),
        scratch_types=[
            pltpu.VMEM((seq_per_w, 128), jnp.int32),
            pltpu.VMEM((NBUF, seq_len, EMBED), jnp.float32),
            pltpu.SemaphoreType.DMA((NBUF,)),
            pltpu.SemaphoreType.DMA((NBUF,)),
        ],
    )
    outs = [call(tok_cubes[k], table) for k in range(NCHUNK)]
    return jnp.concatenate(outs, axis=0)


# speedup vs baseline: 1.0107x; 1.0107x over previous
import jax
import jax.numpy as jnp
from jax import lax
from jax.experimental import pallas as pl
from jax.experimental.pallas import tpu as pltpu
from jax.experimental.pallas import tpu_sc as plsc

NC = 2
NS = 16
NW = NC * NS

EMBED = 128
NBUF = 8


def _embed_body(tok_hbm, table_hbm, out_hbm, idx_v, bufs, gsem, wsem,
                seq_per_w, seq_len):
    wid = lax.axis_index("s") * NC + lax.axis_index("c")
    pltpu.sync_copy(tok_hbm.at[wid], idx_v)
    seq0 = wid * seq_per_w

    def g_copy(j, b):
        return pltpu.make_async_copy(
            table_hbm.at[idx_v.at[j, pl.ds(0, seq_len)]], bufs.at[b],
            gsem.at[b])

    def w_copy(j, b):
        return pltpu.make_async_copy(
            bufs.at[b], out_hbm.at[seq0 + j], wsem.at[b])

    for b in range(NBUF):
        g_copy(b, b).start()

    @pl.loop(0, seq_per_w, step=NBUF)
    def _(j0):
        for b in range(NBUF):
            j = j0 + b
            g_copy(j, b).wait()
            w_copy(j, b).start()

            @pl.when(j + NBUF < seq_per_w)
            def _():
                w_copy(j, b).wait()
                g_copy(j + NBUF, b).start()

    for b in range(NBUF):
        w_copy(seq_per_w - NBUF + b, b).wait()


NCHUNK = 4


def kernel(tokens, table):
    n_seq, seq_len = tokens.shape
    assert n_seq % (NW * NCHUNK) == 0
    ch_seq = n_seq // NCHUNK
    seq_per_w = ch_seq // NW
    assert seq_per_w % NBUF == 0
    tok_pad = jnp.zeros((n_seq, 128), jnp.int32)
    tok_pad = lax.dynamic_update_slice(
        tok_pad, tokens.astype(jnp.int32), (0, 0))
    tok_cubes = tok_pad.reshape(NCHUNK, NW, seq_per_w, 128)

    mesh = plsc.VectorSubcoreMesh(
        core_axis_name="c", subcore_axis_name="s",
        num_cores=NC, num_subcores=NS)

    def body(tok_hbm, table_hbm, out_hbm, idx_v, bufs, gsem, wsem):
        _embed_body(tok_hbm, table_hbm, out_hbm, idx_v, bufs, gsem, wsem,
                    seq_per_w, seq_len)

    call = pl.kernel(
        body,
        out_type=jax.ShapeDtypeStruct((ch_seq, seq_len, EMBED), jnp.float32),
        mesh=mesh,
        compiler_params=pltpu.CompilerParams(use_tc_tiling_on_sc=True),
        scratch_types=[
            pltpu.VMEM((seq_per_w, 128), jnp.int32),
            pltpu.VMEM((NBUF, seq_len, EMBED), jnp.float32),
            pltpu.SemaphoreType.DMA((NBUF,)),
            pltpu.SemaphoreType.DMA((NBUF,)),
        ],
    )
    outs = [call(tok_cubes[k], table) for k in range(NCHUNK)]
    out = jnp.zeros((n_seq, seq_len, EMBED), jnp.float32)
    for k in range(NCHUNK):
        out = lax.dynamic_update_slice(out, outs[k], (k * ch_seq, 0, 0))
    return out


# speedup vs baseline: 1.7799x; 1.7611x over previous
import jax
import jax.numpy as jnp
from jax import lax
from jax.experimental import pallas as pl
from jax.experimental.pallas import tpu as pltpu
from jax.experimental.pallas import tpu_sc as plsc

NC = 2
NS = 16
NW = NC * NS

EMBED = 128
NBUF = 8


def _embed_body(tok_hbm, table_hbm, out_hbm, idx_v, bufs, gsem, wsem,
                seq_per_w, seq_len):
    wid = lax.axis_index("s") * NC + lax.axis_index("c")
    pltpu.sync_copy(tok_hbm.at[wid], idx_v)
    seq0 = wid * seq_per_w

    def g_copy(j, b):
        return pltpu.make_async_copy(
            table_hbm.at[idx_v.at[j, pl.ds(0, seq_len)]], bufs.at[b],
            gsem.at[b])

    def w_copy(j, b):
        return pltpu.make_async_copy(
            bufs.at[b], out_hbm.at[seq0 + j], wsem.at[b])

    for b in range(NBUF):
        g_copy(b, b).start()

    @pl.loop(0, seq_per_w, step=NBUF)
    def _(j0):
        for b in range(NBUF):
            j = j0 + b
            g_copy(j, b).wait()
            w_copy(j, b).start()

            @pl.when(j + NBUF < seq_per_w)
            def _():
                w_copy(j, b).wait()
                g_copy(j + NBUF, b).start()

    for b in range(NBUF):
        w_copy(seq_per_w - NBUF + b, b).wait()


def kernel(tokens, table):
    n_seq, seq_len = tokens.shape
    assert n_seq % NW == 0
    seq_per_w = n_seq // NW
    assert seq_per_w % NBUF == 0
    tok_pad = jnp.zeros((n_seq, 128), jnp.int32)
    tok_pad = lax.dynamic_update_slice(
        tok_pad, tokens.astype(jnp.int32), (0, 0))
    tok_cube = tok_pad.reshape(NW, seq_per_w, 128)

    mesh = plsc.VectorSubcoreMesh(
        core_axis_name="c", subcore_axis_name="s",
        num_cores=NC, num_subcores=NS)

    def body(tok_hbm, table_hbm, out_hbm, idx_v, bufs, gsem, wsem):
        _embed_body(tok_hbm, table_hbm, out_hbm, idx_v, bufs, gsem, wsem,
                    seq_per_w, seq_len)

    out = pl.kernel(
        body,
        out_type=jax.ShapeDtypeStruct((n_seq, seq_len, EMBED), jnp.float32),
        mesh=mesh,
        compiler_params=pltpu.CompilerParams(use_tc_tiling_on_sc=True),
        scratch_types=[
            pltpu.VMEM((seq_per_w, 128), jnp.int32),
            pltpu.VMEM((NBUF, seq_len, EMBED), jnp.float32),
            pltpu.SemaphoreType.DMA((NBUF,)),
            pltpu.SemaphoreType.DMA((NBUF,)),
        ],
    )(tok_cube, table)
    return out
